# baseline (speedup 1.0000x reference)
import functools

import jax
import jax.numpy as jnp
from jax import lax
from jax.experimental import pallas as pl
from jax.experimental.pallas import tpu as pltpu
from jax.experimental.pallas import tpu_sc as plsc

N = 10000
E = 320000
F = 128
C = 40
YT = 0.5
KT = 1.0

NCORES = 2
NSUB = 16
NTILES = NCORES * NSUB
CHUNK = 128
NCHUNKS = 80
NPAIRS = NCHUNKS // 2
EPAD = NTILES * NCHUNKS * CHUNK
DUMP_ROW = N
SPM_ROWS = 10240
ZROWS = SPM_ROWS // NSUB

_SC_MESH = plsc.VectorSubcoreMesh(core_axis_name="c", subcore_axis_name="s")


@functools.partial(
    pl.kernel,
    mesh=_SC_MESH,
    out_type=jax.ShapeDtypeStruct((NCORES, SPM_ROWS, F), jnp.float32),
    scratch_types=[
        pltpu.VMEM((2, 1, CHUNK), jnp.int32),
        pltpu.VMEM((NCHUNKS, CHUNK), jnp.int32),
        pltpu.VMEM((CHUNK, F), jnp.float32),
        pltpu.VMEM((CHUNK, F), jnp.float32),
        pltpu.VMEM_SHARED((SPM_ROWS, F), jnp.float32),
        pltpu.SemaphoreType.DMA,
        pltpu.SemaphoreType.DMA,
        pltpu.SemaphoreType.DMA,
        pltpu.SemaphoreType.DMA,
    ],
)
def _sc_spmm(h_hbm, src_hbm, dst_hbm, zeros_hbm, out_hbm,
             ring, dst_v, rows0, rows1, agg_s, isem0, isem1, gsem0, gsem1):
    cid = lax.axis_index("c")
    sid = lax.axis_index("s")
    wid = cid * NSUB + sid
    pltpu.sync_copy(zeros_hbm, agg_s.at[pl.ds(sid * ZROWS, ZROWS)])
    pltpu.sync_copy(dst_hbm.at[wid], dst_v)
    pltpu.async_copy(src_hbm.at[wid, 0], ring.at[0], isem0)
    pltpu.async_copy(src_hbm.at[wid, 1], ring.at[1], isem1)
    plsc.subcore_barrier()

    pltpu.make_async_copy(src_hbm.at[wid, 0], ring.at[0], isem0).wait()
    pltpu.async_copy(h_hbm.at[ring.at[0, 0]], rows0, gsem0)

    def body(i, carry):
        j0 = 2 * i
        pltpu.make_async_copy(src_hbm.at[wid, 0], ring.at[1], isem1).wait()
        pltpu.async_copy(h_hbm.at[ring.at[1, 0]], rows1, gsem1)
        pltpu.make_async_copy(h_hbm.at[ring.at[0, 0]], rows0, gsem0).wait()

        @pl.when(i < NPAIRS - 1)
        def _():
            pltpu.async_copy(src_hbm.at[wid, j0 + 2], ring.at[0], isem0)

        pltpu.sync_copy(rows0, agg_s.at[dst_v.at[j0]], add=True)
        pltpu.make_async_copy(h_hbm.at[ring.at[1, 0]], rows1, gsem1).wait()

        @pl.when(i < NPAIRS - 1)
        def _():
            pltpu.async_copy(src_hbm.at[wid, j0 + 3], ring.at[1], isem1)
            pltpu.make_async_copy(src_hbm.at[wid, 0], ring.at[0], isem0).wait()
            pltpu.async_copy(h_hbm.at[ring.at[0, 0]], rows0, gsem0)

        pltpu.sync_copy(rows1, agg_s.at[dst_v.at[j0 + 1]], add=True)
        return carry

    lax.fori_loop(0, NPAIRS, body, 0)
    plsc.subcore_barrier()
    base = sid * ZROWS
    off = 0
    while off < ZROWS:
        step = min(CHUNK, ZROWS - off)
        pltpu.sync_copy(agg_s.at[pl.ds(base + off, step)],
                        rows0.at[pl.ds(0, step)])
        pltpu.sync_copy(rows0.at[pl.ds(0, step)],
                        out_hbm.at[cid, pl.ds(base + off, step)])
        off += step


@functools.partial(
    pl.kernel,
    mesh=_SC_MESH,
    out_type=jax.ShapeDtypeStruct((NCORES, SPM_ROWS, F), jnp.float32),
    scratch_types=[
        pltpu.VMEM((NCHUNKS, CHUNK), jnp.int32),
        pltpu.VMEM((CHUNK, F), jnp.float32),
        pltpu.VMEM_SHARED((SPM_ROWS, F), jnp.float32),
    ],
)
def _sc_deg(dst_hbm, ones_hbm, zeros_hbm, out_hbm,
            dst_v, ones_v, deg_s):
    cid = lax.axis_index("c")
    sid = lax.axis_index("s")
    wid = cid * NSUB + sid
    pltpu.sync_copy(zeros_hbm, deg_s.at[pl.ds(sid * ZROWS, ZROWS)])
    pltpu.sync_copy(ones_hbm, ones_v)
    pltpu.sync_copy(dst_hbm.at[wid], dst_v)
    plsc.subcore_barrier()

    def body(j, carry):
        pltpu.sync_copy(ones_v, deg_s.at[dst_v.at[j]], add=True)
        return carry

    lax.fori_loop(0, NCHUNKS, body, 0)
    plsc.subcore_barrier()
    base = sid * ZROWS
    off = 0
    while off < ZROWS:
        step = min(CHUNK, ZROWS - off)
        pltpu.sync_copy(deg_s.at[pl.ds(base + off, step)],
                        ones_v.at[pl.ds(0, step)])
        pltpu.sync_copy(ones_v.at[pl.ds(0, step)],
                        out_hbm.at[cid, pl.ds(base + off, step)])
        off += step


NPAD = SPM_ROWS
BLK = 1024


def _layer_body(agg_ref, deg_ref, w_ref, x_ref, h1_ref, o_ref):
    z = agg_ref[0] + agg_ref[1]
    d = deg_ref[0, :, 0:1] + deg_ref[1, :, 0:1]
    scale = YT / jnp.maximum(d, 1.0)
    y = jnp.dot(z, w_ref[...], preferred_element_type=jnp.float32)
    o_ref[...] = jnp.maximum(y * scale + KT * x_ref[...] - h1_ref[...], 0.0)


_tc_layer = pl.pallas_call(
    _layer_body,
    grid=(NPAD // BLK,),
    in_specs=[
        pl.BlockSpec((NCORES, BLK, F), lambda i: (0, i, 0)),
        pl.BlockSpec((NCORES, BLK, F), lambda i: (0, i, 0)),
        pl.BlockSpec((F, F), lambda i: (0, 0)),
        pl.BlockSpec((BLK, F), lambda i: (i, 0)),
        pl.BlockSpec((BLK, F), lambda i: (i, 0)),
    ],
    out_specs=pl.BlockSpec((BLK, F), lambda i: (i, 0)),
    out_shape=jax.ShapeDtypeStruct((NPAD, F), jnp.float32),
)


def _out_body(h_ref, w_ref, b_ref, o_ref):
    logits = jnp.dot(h_ref[...], w_ref[...],
                     preferred_element_type=jnp.float32) + b_ref[...]
    col = lax.broadcasted_iota(jnp.int32, logits.shape, 1)
    valid = col < C
    masked = jnp.where(valid, logits, -jnp.inf)
    m = jnp.max(masked, axis=1, keepdims=True)
    e = jnp.where(valid, jnp.exp(masked - m), 0.0)
    lse = jnp.log(jnp.sum(e, axis=1, keepdims=True)) + m
    o_ref[...] = logits - lse


_tc_out = pl.pallas_call(
    _out_body,
    grid=(NPAD // BLK,),
    in_specs=[
        pl.BlockSpec((BLK, F), lambda i: (i, 0)),
        pl.BlockSpec((F, 128), lambda i: (0, 0)),
        pl.BlockSpec((1, 128), lambda i: (0, 0)),
    ],
    out_specs=pl.BlockSpec((BLK, 128), lambda i: (i, 0)),
    out_shape=jax.ShapeDtypeStruct((NPAD, 128), jnp.float32),
)


def kernel(X, edge_index, Ws, W_out, b_out):
    key = edge_index[0] * 16384 + edge_index[1]
    key = jnp.sort(key)
    src = (key >> 14).astype(jnp.int32)
    dst = (key & 16383).astype(jnp.int32)
    pad = EPAD - E
    src_r = jnp.concatenate(
        [src, jnp.zeros((pad,), jnp.int32)]).reshape(NTILES, NCHUNKS, 1, CHUNK)
    pad_dst = DUMP_ROW + jnp.arange(pad, dtype=jnp.int32) % (SPM_ROWS - N)
    dst_r = jnp.concatenate([dst, pad_dst]).reshape(NTILES, NCHUNKS, CHUNK)
    zeros_f = jnp.zeros((ZROWS, F), jnp.float32)
    ones_d = jnp.ones((CHUNK, F), jnp.float32)
    Xp = jnp.concatenate([X, jnp.zeros((NPAD - N, F), jnp.float32)])

    deg_parts = _sc_deg(dst_r, ones_d, zeros_f)

    def prop(H):
        return _sc_spmm(H, src_r, dst_r, zeros_f)

    Ha = _tc_layer(prop(Xp), deg_parts, Ws[0], Xp, Xp)
    Hb = _tc_layer(prop(Ha), deg_parts, Ws[0], Xp, Xp)
    Hc = _tc_layer(prop(Hb), deg_parts, Ws[1], Xp, Xp)
    Hd = _tc_layer(prop(Hc), deg_parts, Ws[2], Xp, Hb)
    He = _tc_layer(prop(Hd), deg_parts, Ws[3], Xp, Hc)

    W_p = jnp.zeros((F, 128), jnp.float32).at[:, :C].set(W_out)
    b_p = jnp.zeros((1, 128), jnp.float32).at[0, :C].set(b_out)
    out = _tc_out(He, W_p, b_p)
    return out[:N, :C]


# speedup vs baseline: 1.1634x; 1.1634x over previous
import functools

import jax
import jax.numpy as jnp
from jax import lax
from jax.experimental import pallas as pl
from jax.experimental.pallas import tpu as pltpu
from jax.experimental.pallas import tpu_sc as plsc

N = 10000
E = 320000
F = 128
C = 40
YT = 0.5
KT = 1.0

NCORES = 2
NSUB = 16
NTILES = NCORES * NSUB
CHUNK = 128
NCHUNKS = 80
NPAIRS = NCHUNKS // 2
EPAD = NTILES * NCHUNKS * CHUNK
DUMP_ROW = N
SPM_ROWS = 10240
ZROWS = SPM_ROWS // NSUB

_SC_MESH = plsc.VectorSubcoreMesh(core_axis_name="c", subcore_axis_name="s")


@functools.partial(
    pl.kernel,
    mesh=_SC_MESH,
    out_type=jax.ShapeDtypeStruct((NCORES, SPM_ROWS, F), jnp.float32),
    scratch_types=[
        pltpu.VMEM((2, 1, CHUNK), jnp.int32),
        pltpu.VMEM((NCHUNKS, CHUNK), jnp.int32),
        pltpu.VMEM((CHUNK, F), jnp.float32),
        pltpu.VMEM((CHUNK, F), jnp.float32),
        pltpu.VMEM_SHARED((SPM_ROWS, F), jnp.float32),
        pltpu.SemaphoreType.DMA,
        pltpu.SemaphoreType.DMA,
        pltpu.SemaphoreType.DMA,
        pltpu.SemaphoreType.DMA,
        pltpu.SemaphoreType.DMA,
        pltpu.SemaphoreType.DMA,
    ],
)
def _sc_spmm(h_hbm, src_hbm, dst_hbm, zeros_hbm, out_hbm,
             ring, dst_v, rows0, rows1, agg_s,
             isem0, isem1, ga0, ga1, gb0, gb1):
    cid = lax.axis_index("c")
    sid = lax.axis_index("s")
    wid = cid * NSUB + sid
    HC = CHUNK // 2

    def gather2(slot, buf, s0, s1):
        pltpu.async_copy(h_hbm.at[ring.at[slot, 0, pl.ds(0, HC)]],
                         buf.at[pl.ds(0, HC)], s0)
        pltpu.async_copy(h_hbm.at[ring.at[slot, 0, pl.ds(HC, HC)]],
                         buf.at[pl.ds(HC, HC)], s1)

    def wait2(slot, buf, s0, s1):
        pltpu.make_async_copy(h_hbm.at[ring.at[slot, 0, pl.ds(0, HC)]],
                              buf.at[pl.ds(0, HC)], s0).wait()
        pltpu.make_async_copy(h_hbm.at[ring.at[slot, 0, pl.ds(HC, HC)]],
                              buf.at[pl.ds(HC, HC)], s1).wait()

    pltpu.sync_copy(zeros_hbm, agg_s.at[pl.ds(sid * ZROWS, ZROWS)])
    pltpu.sync_copy(dst_hbm.at[wid], dst_v)
    pltpu.async_copy(src_hbm.at[wid, 0], ring.at[0], isem0)
    pltpu.async_copy(src_hbm.at[wid, 1], ring.at[1], isem1)
    plsc.subcore_barrier()

    pltpu.make_async_copy(src_hbm.at[wid, 0], ring.at[0], isem0).wait()
    gather2(0, rows0, ga0, ga1)

    def body(i, carry):
        j0 = 2 * i
        pltpu.make_async_copy(src_hbm.at[wid, 0], ring.at[1], isem1).wait()
        gather2(1, rows1, gb0, gb1)
        wait2(0, rows0, ga0, ga1)

        @pl.when(i < NPAIRS - 1)
        def _():
            pltpu.async_copy(src_hbm.at[wid, j0 + 2], ring.at[0], isem0)

        pltpu.sync_copy(rows0, agg_s.at[dst_v.at[j0]], add=True)

        @pl.when(i < NPAIRS - 1)
        def _():
            pltpu.make_async_copy(src_hbm.at[wid, 0], ring.at[0], isem0).wait()
            gather2(0, rows0, ga0, ga1)

        wait2(1, rows1, gb0, gb1)

        @pl.when(i < NPAIRS - 1)
        def _():
            pltpu.async_copy(src_hbm.at[wid, j0 + 3], ring.at[1], isem1)

        pltpu.sync_copy(rows1, agg_s.at[dst_v.at[j0 + 1]], add=True)
        return carry

    lax.fori_loop(0, NPAIRS, body, 0)
    plsc.subcore_barrier()
    base = sid * ZROWS
    off = 0
    while off < ZROWS:
        step = min(CHUNK, ZROWS - off)
        pltpu.sync_copy(agg_s.at[pl.ds(base + off, step)],
                        rows0.at[pl.ds(0, step)])
        pltpu.sync_copy(rows0.at[pl.ds(0, step)],
                        out_hbm.at[cid, pl.ds(base + off, step)])
        off += step


@functools.partial(
    pl.kernel,
    mesh=_SC_MESH,
    out_type=jax.ShapeDtypeStruct((NCORES, SPM_ROWS, F), jnp.float32),
    scratch_types=[
        pltpu.VMEM((NCHUNKS, CHUNK), jnp.int32),
        pltpu.VMEM((CHUNK, F), jnp.float32),
        pltpu.VMEM_SHARED((SPM_ROWS, F), jnp.float32),
    ],
)
def _sc_deg(dst_hbm, ones_hbm, zeros_hbm, out_hbm,
            dst_v, ones_v, deg_s):
    cid = lax.axis_index("c")
    sid = lax.axis_index("s")
    wid = cid * NSUB + sid
    pltpu.sync_copy(zeros_hbm, deg_s.at[pl.ds(sid * ZROWS, ZROWS)])
    pltpu.sync_copy(ones_hbm, ones_v)
    pltpu.sync_copy(dst_hbm.at[wid], dst_v)
    plsc.subcore_barrier()

    def body(j, carry):
        pltpu.sync_copy(ones_v, deg_s.at[dst_v.at[j]], add=True)
        return carry

    lax.fori_loop(0, NCHUNKS, body, 0)
    plsc.subcore_barrier()
    base = sid * ZROWS
    off = 0
    while off < ZROWS:
        step = min(CHUNK, ZROWS - off)
        pltpu.sync_copy(deg_s.at[pl.ds(base + off, step)],
                        ones_v.at[pl.ds(0, step)])
        pltpu.sync_copy(ones_v.at[pl.ds(0, step)],
                        out_hbm.at[cid, pl.ds(base + off, step)])
        off += step


NPAD = SPM_ROWS
BLK = 1024


def _layer_body(agg_ref, deg_ref, w_ref, x_ref, h1_ref, o_ref):
    z = agg_ref[0] + agg_ref[1]
    d = deg_ref[0, :, 0:1] + deg_ref[1, :, 0:1]
    scale = YT / jnp.maximum(d, 1.0)
    y = jnp.dot(z, w_ref[...], preferred_element_type=jnp.float32)
    o_ref[...] = jnp.maximum(y * scale + KT * x_ref[...] - h1_ref[...], 0.0)


_tc_layer = pl.pallas_call(
    _layer_body,
    grid=(NPAD // BLK,),
    in_specs=[
        pl.BlockSpec((NCORES, BLK, F), lambda i: (0, i, 0)),
        pl.BlockSpec((NCORES, BLK, F), lambda i: (0, i, 0)),
        pl.BlockSpec((F, F), lambda i: (0, 0)),
        pl.BlockSpec((BLK, F), lambda i: (i, 0)),
        pl.BlockSpec((BLK, F), lambda i: (i, 0)),
    ],
    out_specs=pl.BlockSpec((BLK, F), lambda i: (i, 0)),
    out_shape=jax.ShapeDtypeStruct((NPAD, F), jnp.float32),
)


def _out_body(h_ref, w_ref, b_ref, o_ref):
    logits = jnp.dot(h_ref[...], w_ref[...],
                     preferred_element_type=jnp.float32) + b_ref[...]
    col = lax.broadcasted_iota(jnp.int32, logits.shape, 1)
    valid = col < C
    masked = jnp.where(valid, logits, -jnp.inf)
    m = jnp.max(masked, axis=1, keepdims=True)
    e = jnp.where(valid, jnp.exp(masked - m), 0.0)
    lse = jnp.log(jnp.sum(e, axis=1, keepdims=True)) + m
    o_ref[...] = logits - lse


_tc_out = pl.pallas_call(
    _out_body,
    grid=(NPAD // BLK,),
    in_specs=[
        pl.BlockSpec((BLK, F), lambda i: (i, 0)),
        pl.BlockSpec((F, 128), lambda i: (0, 0)),
        pl.BlockSpec((1, 128), lambda i: (0, 0)),
    ],
    out_specs=pl.BlockSpec((BLK, 128), lambda i: (i, 0)),
    out_shape=jax.ShapeDtypeStruct((NPAD, 128), jnp.float32),
)


def kernel(X, edge_index, Ws, W_out, b_out):
    src = edge_index[0]
    dst = edge_index[1]
    pad = EPAD - E
    src_r = jnp.concatenate(
        [src, jnp.zeros((pad,), jnp.int32)]).reshape(NTILES, NCHUNKS, 1, CHUNK)
    pad_dst = DUMP_ROW + jnp.arange(pad, dtype=jnp.int32) % (SPM_ROWS - N)
    dst_r = jnp.concatenate([dst, pad_dst]).reshape(NTILES, NCHUNKS, CHUNK)
    zeros_f = jnp.zeros((ZROWS, F), jnp.float32)
    ones_d = jnp.ones((CHUNK, F), jnp.float32)
    Xp = jnp.concatenate([X, jnp.zeros((NPAD - N, F), jnp.float32)])

    deg_parts = _sc_deg(dst_r, ones_d, zeros_f)

    def prop(H):
        return _sc_spmm(H, src_r, dst_r, zeros_f)

    Ha = _tc_layer(prop(Xp), deg_parts, Ws[0], Xp, Xp)
    Hb = _tc_layer(prop(Ha), deg_parts, Ws[0], Xp, Xp)
    Hc = _tc_layer(prop(Hb), deg_parts, Ws[1], Xp, Xp)
    Hd = _tc_layer(prop(Hc), deg_parts, Ws[2], Xp, Hb)
    He = _tc_layer(prop(Hd), deg_parts, Ws[3], Xp, Hc)

    W_p = jnp.zeros((F, 128), jnp.float32).at[:, :C].set(W_out)
    b_p = jnp.zeros((1, 128), jnp.float32).at[0, :C].set(b_out)
    out = _tc_out(He, W_p, b_p)
    return out[:N, :C]


# speedup vs baseline: 1.7750x; 1.5258x over previous
import functools

import jax
import jax.numpy as jnp
from jax import lax
from jax.experimental import pallas as pl
from jax.experimental.pallas import tpu as pltpu
from jax.experimental.pallas import tpu_sc as plsc

N = 10000
E = 320000
F = 128
C = 40
YT = 0.5
KT = 1.0

NCORES = 2
NSUB = 16
NTILES = NCORES * NSUB
CHUNK = 128
BIG_CID = 0
NCH_BIG = 104
NCH_SMALL = 54
NPB = NCH_BIG // 2
NPS = NCH_SMALL // 2
E_BIG = NSUB * NCH_BIG * CHUNK
E_SMALL = NSUB * NCH_SMALL * CHUNK
EPAD = E_BIG + E_SMALL
DUMP_ROW = N
SPM_ROWS = 10240
ZROWS = SPM_ROWS // NSUB

_SC_MESH = plsc.VectorSubcoreMesh(core_axis_name="c", subcore_axis_name="s")


@functools.partial(
    pl.kernel,
    mesh=_SC_MESH,
    out_type=jax.ShapeDtypeStruct((NCORES, SPM_ROWS, F), jnp.float32),
    scratch_types=[
        pltpu.VMEM((2, 1, CHUNK), jnp.int32),
        pltpu.VMEM((NCH_BIG, CHUNK), jnp.int32),
        pltpu.VMEM((CHUNK, F), jnp.float32),
        pltpu.VMEM((CHUNK, F), jnp.float32),
        pltpu.VMEM_SHARED((SPM_ROWS, F), jnp.float32),
        pltpu.SemaphoreType.DMA,
        pltpu.SemaphoreType.DMA,
        pltpu.SemaphoreType.DMA,
        pltpu.SemaphoreType.DMA,
    ],
)
def _sc_spmm(h_hbm, src_hbm, dst_hbm, zeros_hbm, out_hbm,
             ring, dst_v, rows0, rows1, agg_s,
             isem0, isem1, gsem0, gsem1):
    cid = lax.axis_index("c")
    sid = lax.axis_index("s")
    wid = cid * NSUB + sid
    pltpu.sync_copy(zeros_hbm, agg_s.at[pl.ds(sid * ZROWS, ZROWS)])
    pltpu.sync_copy(dst_hbm.at[wid], dst_v)
    pltpu.async_copy(src_hbm.at[wid, 0], ring.at[0], isem0)
    pltpu.async_copy(src_hbm.at[wid, 1], ring.at[1], isem1)
    plsc.subcore_barrier()

    @pl.when(cid == BIG_CID)
    def _big():
        pltpu.make_async_copy(src_hbm.at[wid, 0], ring.at[0], isem0).wait()
        pltpu.async_copy(h_hbm.at[ring.at[0, 0]], rows0, gsem0)

        def body(i, carry):
            j0 = 2 * i
            pltpu.make_async_copy(src_hbm.at[wid, 0], ring.at[1], isem1).wait()
            pltpu.async_copy(h_hbm.at[ring.at[1, 0]], rows1, gsem1)
            pltpu.make_async_copy(h_hbm.at[ring.at[0, 0]], rows0, gsem0).wait()

            @pl.when(i < NPB - 1)
            def _():
                pltpu.async_copy(src_hbm.at[wid, j0 + 2], ring.at[0], isem0)

            pltpu.sync_copy(rows0, agg_s.at[dst_v.at[j0]], add=True)
            pltpu.make_async_copy(h_hbm.at[ring.at[1, 0]], rows1, gsem1).wait()

            @pl.when(i < NPB - 1)
            def _():
                pltpu.async_copy(src_hbm.at[wid, j0 + 3], ring.at[1], isem1)
                pltpu.make_async_copy(src_hbm.at[wid, 0], ring.at[0],
                                      isem0).wait()
                pltpu.async_copy(h_hbm.at[ring.at[0, 0]], rows0, gsem0)

            pltpu.sync_copy(rows1, agg_s.at[dst_v.at[j0 + 1]], add=True)
            return carry

        lax.fori_loop(0, NPB, body, 0)

    @pl.when(cid != BIG_CID)
    def _small():
        def body(i, carry):
            j0 = 2 * i
            pltpu.make_async_copy(src_hbm.at[wid, 0], ring.at[0], isem0).wait()
            pltpu.async_copy(h_hbm.at[ring.at[0, 0]], rows0, gsem0)
            pltpu.make_async_copy(h_hbm.at[ring.at[0, 0]], rows0, gsem0).wait()

            @pl.when(i < NPS - 1)
            def _():
                pltpu.async_copy(src_hbm.at[wid, j0 + 2], ring.at[0], isem0)

            pltpu.sync_copy(rows0, agg_s.at[dst_v.at[j0]], add=True)
            pltpu.make_async_copy(src_hbm.at[wid, 0], ring.at[1], isem1).wait()
            pltpu.async_copy(h_hbm.at[ring.at[1, 0]], rows1, gsem1)
            pltpu.make_async_copy(h_hbm.at[ring.at[1, 0]], rows1, gsem1).wait()

            @pl.when(i < NPS - 1)
            def _():
                pltpu.async_copy(src_hbm.at[wid, j0 + 3], ring.at[1], isem1)

            pltpu.sync_copy(rows1, agg_s.at[dst_v.at[j0 + 1]], add=True)
            return carry

        lax.fori_loop(0, NPS, body, 0)

    plsc.subcore_barrier()
    base = sid * ZROWS
    for k in range(ZROWS // CHUNK):
        pltpu.sync_copy(agg_s.at[pl.ds(base + k * CHUNK, CHUNK)], rows0)
        pltpu.sync_copy(rows0, out_hbm.at[cid, pl.ds(base + k * CHUNK, CHUNK)])


@functools.partial(
    pl.kernel,
    mesh=_SC_MESH,
    out_type=jax.ShapeDtypeStruct((NCORES, SPM_ROWS, F), jnp.float32),
    scratch_types=[
        pltpu.VMEM((NCH_BIG, CHUNK), jnp.int32),
        pltpu.VMEM((CHUNK, F), jnp.float32),
        pltpu.VMEM_SHARED((SPM_ROWS, F), jnp.float32),
    ],
)
def _sc_deg(dst_hbm, ones_hbm, zeros_hbm, out_hbm,
            dst_v, ones_v, deg_s):
    cid = lax.axis_index("c")
    sid = lax.axis_index("s")
    wid = cid * NSUB + sid
    pltpu.sync_copy(zeros_hbm, deg_s.at[pl.ds(sid * ZROWS, ZROWS)])
    pltpu.sync_copy(ones_hbm, ones_v)
    pltpu.sync_copy(dst_hbm.at[wid], dst_v)
    plsc.subcore_barrier()

    nc = jnp.where(cid == BIG_CID, NCH_BIG, NCH_SMALL)

    def body(j, carry):
        pltpu.sync_copy(ones_v, deg_s.at[dst_v.at[j]], add=True)
        return carry

    lax.fori_loop(0, nc, body, 0)
    plsc.subcore_barrier()
    base = sid * ZROWS
    for k in range(ZROWS // CHUNK):
        pltpu.sync_copy(deg_s.at[pl.ds(base + k * CHUNK, CHUNK)], ones_v)
        pltpu.sync_copy(ones_v,
                        out_hbm.at[cid, pl.ds(base + k * CHUNK, CHUNK)])


NPAD = SPM_ROWS
BLK = 1024


def _layer_body(agg_ref, deg_ref, w_ref, x_ref, h1_ref, o_ref):
    z = agg_ref[0] + agg_ref[1]
    d = deg_ref[0, :, 0:1] + deg_ref[1, :, 0:1]
    scale = YT / jnp.maximum(d, 1.0)
    y = jnp.dot(z, w_ref[...], preferred_element_type=jnp.float32)
    o_ref[...] = jnp.maximum(y * scale + KT * x_ref[...] - h1_ref[...], 0.0)


_tc_layer = pl.pallas_call(
    _layer_body,
    grid=(NPAD // BLK,),
    in_specs=[
        pl.BlockSpec((NCORES, BLK, F), lambda i: (0, i, 0)),
        pl.BlockSpec((NCORES, BLK, F), lambda i: (0, i, 0)),
        pl.BlockSpec((F, F), lambda i: (0, 0)),
        pl.BlockSpec((BLK, F), lambda i: (i, 0)),
        pl.BlockSpec((BLK, F), lambda i: (i, 0)),
    ],
    out_specs=pl.BlockSpec((BLK, F), lambda i: (i, 0)),
    out_shape=jax.ShapeDtypeStruct((NPAD, F), jnp.float32),
)


def _out_body(h_ref, w_ref, b_ref, o_ref):
    logits = jnp.dot(h_ref[...], w_ref[...],
                     preferred_element_type=jnp.float32) + b_ref[...]
    col = lax.broadcasted_iota(jnp.int32, logits.shape, 1)
    valid = col < C
    masked = jnp.where(valid, logits, -jnp.inf)
    m = jnp.max(masked, axis=1, keepdims=True)
    e = jnp.where(valid, jnp.exp(masked - m), 0.0)
    lse = jnp.log(jnp.sum(e, axis=1, keepdims=True)) + m
    o_ref[...] = logits - lse


_tc_out = pl.pallas_call(
    _out_body,
    grid=(NPAD // BLK,),
    in_specs=[
        pl.BlockSpec((BLK, F), lambda i: (i, 0)),
        pl.BlockSpec((F, 128), lambda i: (0, 0)),
        pl.BlockSpec((1, 128), lambda i: (0, 0)),
    ],
    out_specs=pl.BlockSpec((BLK, 128), lambda i: (i, 0)),
    out_shape=jax.ShapeDtypeStruct((NPAD, 128), jnp.float32),
)


def _edge_layout(v, pad_val_fn):
    big = v[:E_BIG].reshape(NSUB, NCH_BIG, CHUNK)
    tail = v[E_BIG:]
    pad = E_SMALL - tail.shape[0]
    small = jnp.concatenate([tail, pad_val_fn(pad)]).reshape(
        NSUB, NCH_SMALL, CHUNK)
    small = jnp.pad(small, ((0, 0), (0, NCH_BIG - NCH_SMALL), (0, 0)))
    parts = (big, small) if BIG_CID == 0 else (small, big)
    return jnp.concatenate(parts, axis=0)


def kernel(X, edge_index, Ws, W_out, b_out):
    src = edge_index[0]
    dst = edge_index[1]
    src_r = _edge_layout(
        src, lambda p: jnp.zeros((p,), jnp.int32)).reshape(
            NTILES, NCH_BIG, 1, CHUNK)
    dst_r = _edge_layout(
        dst, lambda p: DUMP_ROW + jnp.arange(p, dtype=jnp.int32)
        % (SPM_ROWS - N))
    zeros_f = jnp.zeros((ZROWS, F), jnp.float32)
    ones_d = jnp.ones((CHUNK, F), jnp.float32)
    Xp = jnp.concatenate([X, jnp.zeros((NPAD - N, F), jnp.float32)])

    deg_parts = _sc_deg(dst_r, ones_d, zeros_f)

    def prop(H):
        return _sc_spmm(H, src_r, dst_r, zeros_f)

    Ha = _tc_layer(prop(Xp), deg_parts, Ws[0], Xp, Xp)
    Hb = _tc_layer(prop(Ha), deg_parts, Ws[0], Xp, Xp)
    Hc = _tc_layer(prop(Hb), deg_parts, Ws[1], Xp, Xp)
    Hd = _tc_layer(prop(Hc), deg_parts, Ws[2], Xp, Hb)
    He = _tc_layer(prop(Hd), deg_parts, Ws[3], Xp, Hc)

    W_p = jnp.zeros((F, 128), jnp.float32).at[:, :C].set(W_out)
    b_p = jnp.zeros((1, 128), jnp.float32).at[0, :C].set(b_out)
    out = _tc_out(He, W_p, b_p)
    return out[:N, :C]


# speedup vs baseline: 1.8009x; 1.0146x over previous
import functools

import jax
import jax.numpy as jnp
from jax import lax
from jax.experimental import pallas as pl
from jax.experimental.pallas import tpu as pltpu
from jax.experimental.pallas import tpu_sc as plsc

N = 10000
E = 320000
F = 128
C = 40
YT = 0.5
KT = 1.0

NCORES = 2
NSUB = 16
NTILES = NCORES * NSUB
CHUNK = 128
BIG_CID = 1
NCH_BIG = 104
NCH_SMALL = 54
NPB = NCH_BIG // 2
NPS = NCH_SMALL // 2
E_BIG = NSUB * NCH_BIG * CHUNK
E_SMALL = NSUB * NCH_SMALL * CHUNK
EPAD = E_BIG + E_SMALL
DUMP_ROW = N
SPM_ROWS = 10240
ZROWS = SPM_ROWS // NSUB

_SC_MESH = plsc.VectorSubcoreMesh(core_axis_name="c", subcore_axis_name="s")


@functools.partial(
    pl.kernel,
    mesh=_SC_MESH,
    out_type=jax.ShapeDtypeStruct((NCORES, SPM_ROWS, F), jnp.float32),
    scratch_types=[
        pltpu.VMEM((2, 1, CHUNK), jnp.int32),
        pltpu.VMEM((NCH_BIG, CHUNK), jnp.int32),
        pltpu.VMEM((CHUNK, F), jnp.float32),
        pltpu.VMEM((CHUNK, F), jnp.float32),
        pltpu.VMEM_SHARED((SPM_ROWS, F), jnp.float32),
        pltpu.SemaphoreType.DMA,
        pltpu.SemaphoreType.DMA,
        pltpu.SemaphoreType.DMA,
        pltpu.SemaphoreType.DMA,
    ],
)
def _sc_spmm(h_hbm, src_hbm, dst_hbm, zeros_hbm, out_hbm,
             ring, dst_v, rows0, rows1, agg_s,
             isem0, isem1, gsem0, gsem1):
    cid = lax.axis_index("c")
    sid = lax.axis_index("s")
    wid = cid * NSUB + sid
    pltpu.sync_copy(zeros_hbm, agg_s.at[pl.ds(sid * ZROWS, ZROWS)])
    pltpu.sync_copy(dst_hbm.at[wid], dst_v)
    pltpu.async_copy(src_hbm.at[wid, 0], ring.at[0], isem0)
    pltpu.async_copy(src_hbm.at[wid, 1], ring.at[1], isem1)
    plsc.subcore_barrier()

    @pl.when(cid == BIG_CID)
    def _big():
        pltpu.make_async_copy(src_hbm.at[wid, 0], ring.at[0], isem0).wait()
        pltpu.async_copy(h_hbm.at[ring.at[0, 0]], rows0, gsem0)

        def body(i, carry):
            j0 = 2 * i
            pltpu.make_async_copy(src_hbm.at[wid, 0], ring.at[1], isem1).wait()
            pltpu.async_copy(h_hbm.at[ring.at[1, 0]], rows1, gsem1)
            pltpu.make_async_copy(h_hbm.at[ring.at[0, 0]], rows0, gsem0).wait()

            @pl.when(i < NPB - 1)
            def _():
                pltpu.async_copy(src_hbm.at[wid, j0 + 2], ring.at[0], isem0)

            pltpu.sync_copy(rows0, agg_s.at[dst_v.at[j0]], add=True)
            pltpu.make_async_copy(h_hbm.at[ring.at[1, 0]], rows1, gsem1).wait()

            @pl.when(i < NPB - 1)
            def _():
                pltpu.async_copy(src_hbm.at[wid, j0 + 3], ring.at[1], isem1)
                pltpu.make_async_copy(src_hbm.at[wid, 0], ring.at[0],
                                      isem0).wait()
                pltpu.async_copy(h_hbm.at[ring.at[0, 0]], rows0, gsem0)

            pltpu.sync_copy(rows1, agg_s.at[dst_v.at[j0 + 1]], add=True)
            return carry

        lax.fori_loop(0, NPB, body, 0)

    @pl.when(cid != BIG_CID)
    def _small():
        def body(i, carry):
            j0 = 2 * i
            pltpu.make_async_copy(src_hbm.at[wid, 0], ring.at[0], isem0).wait()
            pltpu.async_copy(h_hbm.at[ring.at[0, 0]], rows0, gsem0)
            pltpu.make_async_copy(h_hbm.at[ring.at[0, 0]], rows0, gsem0).wait()

            @pl.when(i < NPS - 1)
            def _():
                pltpu.async_copy(src_hbm.at[wid, j0 + 2], ring.at[0], isem0)

            pltpu.sync_copy(rows0, agg_s.at[dst_v.at[j0]], add=True)
            pltpu.make_async_copy(src_hbm.at[wid, 0], ring.at[1], isem1).wait()
            pltpu.async_copy(h_hbm.at[ring.at[1, 0]], rows1, gsem1)
            pltpu.make_async_copy(h_hbm.at[ring.at[1, 0]], rows1, gsem1).wait()

            @pl.when(i < NPS - 1)
            def _():
                pltpu.async_copy(src_hbm.at[wid, j0 + 3], ring.at[1], isem1)

            pltpu.sync_copy(rows1, agg_s.at[dst_v.at[j0 + 1]], add=True)
            return carry

        lax.fori_loop(0, NPS, body, 0)

    plsc.subcore_barrier()
    base = sid * ZROWS
    for k in range(ZROWS // CHUNK):
        pltpu.sync_copy(agg_s.at[pl.ds(base + k * CHUNK, CHUNK)], rows0)
        pltpu.sync_copy(rows0, out_hbm.at[cid, pl.ds(base + k * CHUNK, CHUNK)])


@functools.partial(
    pl.kernel,
    mesh=_SC_MESH,
    out_type=jax.ShapeDtypeStruct((NCORES, SPM_ROWS, F), jnp.float32),
    scratch_types=[
        pltpu.VMEM((NCH_BIG, CHUNK), jnp.int32),
        pltpu.VMEM((CHUNK, F), jnp.float32),
        pltpu.VMEM_SHARED((SPM_ROWS, F), jnp.float32),
    ],
)
def _sc_deg(dst_hbm, ones_hbm, zeros_hbm, out_hbm,
            dst_v, ones_v, deg_s):
    cid = lax.axis_index("c")
    sid = lax.axis_index("s")
    wid = cid * NSUB + sid
    pltpu.sync_copy(zeros_hbm, deg_s.at[pl.ds(sid * ZROWS, ZROWS)])
    pltpu.sync_copy(ones_hbm, ones_v)
    pltpu.sync_copy(dst_hbm.at[wid], dst_v)
    plsc.subcore_barrier()

    nc = jnp.where(cid == BIG_CID, NCH_BIG, NCH_SMALL)

    def body(j, carry):
        pltpu.sync_copy(ones_v, deg_s.at[dst_v.at[j]], add=True)
        return carry

    lax.fori_loop(0, nc, body, 0)
    plsc.subcore_barrier()
    base = sid * ZROWS
    for k in range(ZROWS // CHUNK):
        pltpu.sync_copy(deg_s.at[pl.ds(base + k * CHUNK, CHUNK)], ones_v)
        pltpu.sync_copy(ones_v,
                        out_hbm.at[cid, pl.ds(base + k * CHUNK, CHUNK)])


NPAD = SPM_ROWS
BLK = 1024


def _layer_body(agg_ref, deg_ref, w_ref, x_ref, h1_ref, o_ref):
    z = agg_ref[0] + agg_ref[1]
    d = deg_ref[0, :, 0:1] + deg_ref[1, :, 0:1]
    scale = YT / jnp.maximum(d, 1.0)
    y = jnp.dot(z, w_ref[...], preferred_element_type=jnp.float32)
    o_ref[...] = jnp.maximum(y * scale + KT * x_ref[...] - h1_ref[...], 0.0)


_tc_layer = pl.pallas_call(
    _layer_body,
    grid=(NPAD // BLK,),
    in_specs=[
        pl.BlockSpec((NCORES, BLK, F), lambda i: (0, i, 0)),
        pl.BlockSpec((NCORES, BLK, F), lambda i: (0, i, 0)),
        pl.BlockSpec((F, F), lambda i: (0, 0)),
        pl.BlockSpec((BLK, F), lambda i: (i, 0)),
        pl.BlockSpec((BLK, F), lambda i: (i, 0)),
    ],
    out_specs=pl.BlockSpec((BLK, F), lambda i: (i, 0)),
    out_shape=jax.ShapeDtypeStruct((NPAD, F), jnp.float32),
)


def _out_body(h_ref, w_ref, b_ref, o_ref):
    logits = jnp.dot(h_ref[...], w_ref[...],
                     preferred_element_type=jnp.float32) + b_ref[...]
    col = lax.broadcasted_iota(jnp.int32, logits.shape, 1)
    valid = col < C
    masked = jnp.where(valid, logits, -jnp.inf)
    m = jnp.max(masked, axis=1, keepdims=True)
    e = jnp.where(valid, jnp.exp(masked - m), 0.0)
    lse = jnp.log(jnp.sum(e, axis=1, keepdims=True)) + m
    o_ref[...] = logits - lse


_tc_out = pl.pallas_call(
    _out_body,
    grid=(NPAD // BLK,),
    in_specs=[
        pl.BlockSpec((BLK, F), lambda i: (i, 0)),
        pl.BlockSpec((F, 128), lambda i: (0, 0)),
        pl.BlockSpec((1, 128), lambda i: (0, 0)),
    ],
    out_specs=pl.BlockSpec((BLK, 128), lambda i: (i, 0)),
    out_shape=jax.ShapeDtypeStruct((NPAD, 128), jnp.float32),
)


def _edge_layout(v, pad_val_fn):
    big = v[:E_BIG].reshape(NSUB, NCH_BIG, CHUNK)
    tail = v[E_BIG:]
    pad = E_SMALL - tail.shape[0]
    small = jnp.concatenate([tail, pad_val_fn(pad)]).reshape(
        NSUB, NCH_SMALL, CHUNK)
    small = jnp.pad(small, ((0, 0), (0, NCH_BIG - NCH_SMALL), (0, 0)))
    parts = (big, small) if BIG_CID == 0 else (small, big)
    return jnp.concatenate(parts, axis=0)


def kernel(X, edge_index, Ws, W_out, b_out):
    src = edge_index[0]
    dst = edge_index[1]
    src_r = _edge_layout(
        src, lambda p: jnp.zeros((p,), jnp.int32)).reshape(
            NTILES, NCH_BIG, 1, CHUNK)
    dst_r = _edge_layout(
        dst, lambda p: DUMP_ROW + jnp.arange(p, dtype=jnp.int32)
        % (SPM_ROWS - N))
    zeros_f = jnp.zeros((ZROWS, F), jnp.float32)
    ones_d = jnp.ones((CHUNK, F), jnp.float32)
    Xp = jnp.concatenate([X, jnp.zeros((NPAD - N, F), jnp.float32)])

    deg_parts = _sc_deg(dst_r, ones_d, zeros_f)

    def prop(H):
        return _sc_spmm(H, src_r, dst_r, zeros_f)

    Ha = _tc_layer(prop(Xp), deg_parts, Ws[0], Xp, Xp)
    Hb = _tc_layer(prop(Ha), deg_parts, Ws[0], Xp, Xp)
    Hc = _tc_layer(prop(Hb), deg_parts, Ws[1], Xp, Xp)
    Hd = _tc_layer(prop(Hc), deg_parts, Ws[2], Xp, Hb)
    He = _tc_layer(prop(Hd), deg_parts, Ws[3], Xp, Hc)

    W_p = jnp.zeros((F, 128), jnp.float32).at[:, :C].set(W_out)
    b_p = jnp.zeros((1, 128), jnp.float32).at[0, :C].set(b_out)
    out = _tc_out(He, W_p, b_p)
    return out[:N, :C]


# speedup vs baseline: 1.9851x; 1.1023x over previous
import functools

import jax
import jax.numpy as jnp
from jax import lax
from jax.experimental import pallas as pl
from jax.experimental.pallas import tpu as pltpu
from jax.experimental.pallas import tpu_sc as plsc

N = 10000
E = 320000
F = 128
C = 40
YT = 0.5
KT = 1.0

NCORES = 2
NSUB = 16
NTILES = NCORES * NSUB
CHUNK = 128
BIG_CID = 1
NCH_BIG = 112
NCH_SMALL = 46
NPB = NCH_BIG // 2
NPS = NCH_SMALL // 2
E_BIG = NSUB * NCH_BIG * CHUNK
E_SMALL = NSUB * NCH_SMALL * CHUNK
EPAD = E_BIG + E_SMALL
DUMP_ROW = N
SPM_ROWS = 10112
ZROWS = SPM_ROWS // NSUB

_SC_MESH = plsc.VectorSubcoreMesh(core_axis_name="c", subcore_axis_name="s")


@functools.partial(
    pl.kernel,
    mesh=_SC_MESH,
    out_type=jax.ShapeDtypeStruct((NCORES, SPM_ROWS, F), jnp.float32),
    scratch_types=[
        pltpu.VMEM((2, 1, CHUNK), jnp.int32),
        pltpu.VMEM((NCH_BIG, CHUNK), jnp.int32),
        pltpu.VMEM((CHUNK, F), jnp.float32),
        pltpu.VMEM((CHUNK, F), jnp.float32),
        pltpu.VMEM_SHARED((SPM_ROWS, F), jnp.float32),
        pltpu.SemaphoreType.DMA,
        pltpu.SemaphoreType.DMA,
        pltpu.SemaphoreType.DMA,
        pltpu.SemaphoreType.DMA,
        pltpu.SemaphoreType.DMA,
        pltpu.SemaphoreType.DMA,
    ],
)
def _sc_spmm(h_hbm, src_hbm, dst_hbm, zeros_hbm, out_hbm,
             ring, dst_v, rows0, rows1, agg_s,
             isem0, isem1, gsem0, gsem1, ssem0, ssem1):
    cid = lax.axis_index("c")
    sid = lax.axis_index("s")
    wid = cid * NSUB + sid
    pltpu.sync_copy(zeros_hbm, agg_s.at[pl.ds(sid * ZROWS, ZROWS)])
    pltpu.sync_copy(dst_hbm.at[wid], dst_v)
    pltpu.async_copy(src_hbm.at[wid, 0], ring.at[0], isem0)
    pltpu.async_copy(src_hbm.at[wid, 1], ring.at[1], isem1)
    plsc.subcore_barrier()

    @pl.when(cid == BIG_CID)
    def _big():
        pltpu.make_async_copy(src_hbm.at[wid, 0], ring.at[0], isem0).wait()
        pltpu.async_copy(h_hbm.at[ring.at[0, 0]], rows0, gsem0)

        def body(i, carry):
            j0 = 2 * i
            pltpu.make_async_copy(src_hbm.at[wid, 0], ring.at[1], isem1).wait()
            pltpu.async_copy(h_hbm.at[ring.at[1, 0]], rows1, gsem1)
            pltpu.make_async_copy(h_hbm.at[ring.at[0, 0]], rows0, gsem0).wait()

            @pl.when(i < NPB - 1)
            def _():
                pltpu.async_copy(src_hbm.at[wid, j0 + 2], ring.at[0], isem0)

            pltpu.sync_copy(rows0, agg_s.at[dst_v.at[j0]], add=True)
            pltpu.make_async_copy(h_hbm.at[ring.at[1, 0]], rows1, gsem1).wait()

            @pl.when(i < NPB - 1)
            def _():
                pltpu.async_copy(src_hbm.at[wid, j0 + 3], ring.at[1], isem1)
                pltpu.make_async_copy(src_hbm.at[wid, 0], ring.at[0],
                                      isem0).wait()
                pltpu.async_copy(h_hbm.at[ring.at[0, 0]], rows0, gsem0)

            pltpu.sync_copy(rows1, agg_s.at[dst_v.at[j0 + 1]], add=True)
            return carry

        lax.fori_loop(0, NPB, body, 0)

    @pl.when(cid != BIG_CID)
    def _small():
        def body(i, carry):
            j0 = 2 * i

            @pl.when(i > 0)
            def _():
                pltpu.make_async_copy(
                    rows0, agg_s.at[dst_v.at[j0]], ssem0).wait()

            pltpu.make_async_copy(src_hbm.at[wid, 0], ring.at[0], isem0).wait()
            pltpu.async_copy(h_hbm.at[ring.at[0, 0]], rows0, gsem0)
            pltpu.make_async_copy(h_hbm.at[ring.at[0, 0]], rows0, gsem0).wait()

            @pl.when(i < NPS - 1)
            def _():
                pltpu.async_copy(src_hbm.at[wid, j0 + 2], ring.at[0], isem0)

            pltpu.async_copy(rows0, agg_s.at[dst_v.at[j0]], ssem0, add=True)

            @pl.when(i > 0)
            def _():
                pltpu.make_async_copy(
                    rows1, agg_s.at[dst_v.at[j0 + 1]], ssem1).wait()

            pltpu.make_async_copy(src_hbm.at[wid, 0], ring.at[1], isem1).wait()
            pltpu.async_copy(h_hbm.at[ring.at[1, 0]], rows1, gsem1)
            pltpu.make_async_copy(h_hbm.at[ring.at[1, 0]], rows1, gsem1).wait()

            @pl.when(i < NPS - 1)
            def _():
                pltpu.async_copy(src_hbm.at[wid, j0 + 3], ring.at[1], isem1)

            pltpu.async_copy(rows1, agg_s.at[dst_v.at[j0 + 1]], ssem1,
                             add=True)
            return carry

        lax.fori_loop(0, NPS, body, 0)
        pltpu.make_async_copy(rows0, agg_s.at[dst_v.at[0]], ssem0).wait()
        pltpu.make_async_copy(rows1, agg_s.at[dst_v.at[1]], ssem1).wait()

    plsc.subcore_barrier()
    base = sid * ZROWS
    off = 0
    while off < ZROWS:
        step = min(CHUNK, ZROWS - off)
        pltpu.sync_copy(agg_s.at[pl.ds(base + off, step)],
                        rows0.at[pl.ds(0, step)])
        pltpu.sync_copy(rows0.at[pl.ds(0, step)],
                        out_hbm.at[cid, pl.ds(base + off, step)])
        off += step


@functools.partial(
    pl.kernel,
    mesh=_SC_MESH,
    out_type=jax.ShapeDtypeStruct((NCORES, SPM_ROWS, F), jnp.float32),
    scratch_types=[
        pltpu.VMEM((NCH_BIG, CHUNK), jnp.int32),
        pltpu.VMEM((CHUNK, F), jnp.float32),
        pltpu.VMEM_SHARED((SPM_ROWS, F), jnp.float32),
    ],
)
def _sc_deg(dst_hbm, ones_hbm, zeros_hbm, out_hbm,
            dst_v, ones_v, deg_s):
    cid = lax.axis_index("c")
    sid = lax.axis_index("s")
    wid = cid * NSUB + sid
    pltpu.sync_copy(zeros_hbm, deg_s.at[pl.ds(sid * ZROWS, ZROWS)])
    pltpu.sync_copy(ones_hbm, ones_v)
    pltpu.sync_copy(dst_hbm.at[wid], dst_v)
    plsc.subcore_barrier()

    nc = jnp.where(cid == BIG_CID, NCH_BIG, NCH_SMALL)

    def body(j, carry):
        pltpu.sync_copy(ones_v, deg_s.at[dst_v.at[j]], add=True)
        return carry

    lax.fori_loop(0, nc, body, 0)
    plsc.subcore_barrier()
    base = sid * ZROWS
    off = 0
    while off < ZROWS:
        step = min(CHUNK, ZROWS - off)
        pltpu.sync_copy(deg_s.at[pl.ds(base + off, step)],
                        ones_v.at[pl.ds(0, step)])
        pltpu.sync_copy(ones_v.at[pl.ds(0, step)],
                        out_hbm.at[cid, pl.ds(base + off, step)])
        off += step


NPAD = SPM_ROWS
BLK = 1264


def _layer_body(agg_ref, deg_ref, w_ref, x_ref, h1_ref, o_ref):
    z = agg_ref[0] + agg_ref[1]
    d = deg_ref[0, :, 0:1] + deg_ref[1, :, 0:1]
    scale = YT / jnp.maximum(d, 1.0)
    y = jnp.dot(z, w_ref[...], preferred_element_type=jnp.float32)
    o_ref[...] = jnp.maximum(y * scale + KT * x_ref[...] - h1_ref[...], 0.0)


_tc_layer = pl.pallas_call(
    _layer_body,
    grid=(NPAD // BLK,),
    in_specs=[
        pl.BlockSpec((NCORES, BLK, F), lambda i: (0, i, 0)),
        pl.BlockSpec((NCORES, BLK, F), lambda i: (0, i, 0)),
        pl.BlockSpec((F, F), lambda i: (0, 0)),
        pl.BlockSpec((BLK, F), lambda i: (i, 0)),
        pl.BlockSpec((BLK, F), lambda i: (i, 0)),
    ],
    out_specs=pl.BlockSpec((BLK, F), lambda i: (i, 0)),
    out_shape=jax.ShapeDtypeStruct((NPAD, F), jnp.float32),
)


def _out_body(h_ref, w_ref, b_ref, o_ref):
    logits = jnp.dot(h_ref[...], w_ref[...],
                     preferred_element_type=jnp.float32) + b_ref[...]
    col = lax.broadcasted_iota(jnp.int32, logits.shape, 1)
    valid = col < C
    masked = jnp.where(valid, logits, -jnp.inf)
    m = jnp.max(masked, axis=1, keepdims=True)
    e = jnp.where(valid, jnp.exp(masked - m), 0.0)
    lse = jnp.log(jnp.sum(e, axis=1, keepdims=True)) + m
    o_ref[...] = logits - lse


_tc_out = pl.pallas_call(
    _out_body,
    grid=(NPAD // BLK,),
    in_specs=[
        pl.BlockSpec((BLK, F), lambda i: (i, 0)),
        pl.BlockSpec((F, 128), lambda i: (0, 0)),
        pl.BlockSpec((1, 128), lambda i: (0, 0)),
    ],
    out_specs=pl.BlockSpec((BLK, 128), lambda i: (i, 0)),
    out_shape=jax.ShapeDtypeStruct((NPAD, 128), jnp.float32),
)


def _edge_layout(v, pad_val_fn):
    big = v[:E_BIG].reshape(NSUB, NCH_BIG, CHUNK)
    tail = v[E_BIG:]
    pad = E_SMALL - tail.shape[0]
    small = jnp.concatenate([tail, pad_val_fn(pad)]).reshape(
        NSUB, NCH_SMALL, CHUNK)
    small = jnp.pad(small, ((0, 0), (0, NCH_BIG - NCH_SMALL), (0, 0)))
    parts = (big, small) if BIG_CID == 0 else (small, big)
    return jnp.concatenate(parts, axis=0)


def kernel(X, edge_index, Ws, W_out, b_out):
    src = edge_index[0]
    dst = edge_index[1]
    src_r = _edge_layout(
        src, lambda p: jnp.zeros((p,), jnp.int32)).reshape(
            NTILES, NCH_BIG, 1, CHUNK)
    dst_r = _edge_layout(
        dst, lambda p: DUMP_ROW + jnp.arange(p, dtype=jnp.int32)
        % (SPM_ROWS - N))
    zeros_f = jnp.zeros((ZROWS, F), jnp.float32)
    ones_d = jnp.ones((CHUNK, F), jnp.float32)
    Xp = jnp.concatenate([X, jnp.zeros((NPAD - N, F), jnp.float32)])

    deg_parts = _sc_deg(dst_r, ones_d, zeros_f)

    def prop(H):
        return _sc_spmm(H, src_r, dst_r, zeros_f)

    Ha = _tc_layer(prop(Xp), deg_parts, Ws[0], Xp, Xp)
    Hb = _tc_layer(prop(Ha), deg_parts, Ws[0], Xp, Xp)
    Hc = _tc_layer(prop(Hb), deg_parts, Ws[1], Xp, Xp)
    Hd = _tc_layer(prop(Hc), deg_parts, Ws[2], Xp, Hb)
    He = _tc_layer(prop(Hd), deg_parts, Ws[3], Xp, Hc)

    W_p = jnp.zeros((F, 128), jnp.float32).at[:, :C].set(W_out)
    b_p = jnp.zeros((1, 128), jnp.float32).at[0, :C].set(b_out)
    out = _tc_out(He, W_p, b_p)
    return out[:N, :C]
